# baseline (speedup 1.0000x reference)
import functools

import jax
import jax.numpy as jnp
from jax import lax
from jax.experimental import pallas as pl
from jax.experimental.pallas import tpu as pltpu
from jax.experimental.pallas import tpu_sc as plsc

B = 16384
D = 64
NC = 2
NS = 16
NW = NC * NS
BPW = B // NW
CHUNK = 128
NCHUNK = BPW // CHUNK
GROUPS = BPW // 16

_mesh = plsc.VectorSubcoreMesh(core_axis_name="c", subcore_axis_name="s")
_params = pltpu.CompilerParams(
    needs_layout_passes=False, use_tc_tiling_on_sc=False,
    disable_bounds_checks=True)


@functools.partial(
    pl.kernel,
    mesh=_mesh,
    compiler_params=_params,
    out_type=jax.ShapeDtypeStruct((B, D), jnp.float32),
    scratch_types=[
        pltpu.VMEM((BPW,), jnp.int32),
        pltpu.VMEM((BPW,), jnp.int32),
        pltpu.VMEM((BPW, D), jnp.float32),
        pltpu.VMEM((BPW, D), jnp.float32),
        pltpu.SemaphoreType.DMA,
    ],
)
def _gather_hr_kernel(heads_hbm, rel_hbm, ent_hbm, relt_hbm,
                      hr_hbm, h_idx, r_idx, h_rows, r_rows, sem):
    wid = lax.axis_index("s") * NC + lax.axis_index("c")
    base = wid * BPW

    pltpu.sync_copy(heads_hbm.at[pl.ds(base, BPW)], h_idx)
    pltpu.sync_copy(rel_hbm.at[pl.ds(base, BPW)], r_idx)

    copies = []
    for c in range(NCHUNK):
        sl = pl.ds(c * CHUNK, CHUNK)
        copies.append(pltpu.async_copy(ent_hbm.at[h_idx.at[sl]], h_rows.at[sl], sem))
        copies.append(pltpu.async_copy(relt_hbm.at[r_idx.at[sl]], r_rows.at[sl], sem))
    for cp in copies:
        cp.wait()

    def add_body(b, carry):
        for c in range(D // 16):
            sl = pl.ds(c * 16, 16)
            h_rows[b, sl] = h_rows[b, sl] + r_rows[b, sl]
        return carry

    lax.fori_loop(0, BPW, add_body, 0)
    pltpu.sync_copy(h_rows, hr_hbm.at[pl.ds(base, BPW)])


@functools.partial(
    pl.kernel,
    mesh=_mesh,
    compiler_params=_params,
    out_type=jax.ShapeDtypeStruct((B,), jnp.float32),
    scratch_types=[
        pltpu.VMEM((BPW,), jnp.int32),
        pltpu.VMEM((BPW, D), jnp.float32),
        pltpu.VMEM((BPW, D), jnp.float32),
        pltpu.VMEM((BPW,), jnp.float32),
        pltpu.VMEM((256,), jnp.float32),
        pltpu.SemaphoreType.DMA,
    ],
)
def _score_kernel(tails_hbm, ent_hbm, hr_hbm, out_hbm,
                  t_idx, t_rows, hr_rows, out_v, tbuf, sem):
    wid = lax.axis_index("s") * NC + lax.axis_index("c")
    base = wid * BPW

    pltpu.sync_copy(tails_hbm.at[pl.ds(base, BPW)], t_idx)
    cp0 = pltpu.async_copy(hr_hbm.at[pl.ds(base, BPW)], hr_rows, sem)

    copies = [cp0]
    for c in range(NCHUNK):
        sl = pl.ds(c * CHUNK, CHUNK)
        copies.append(pltpu.async_copy(ent_hbm.at[t_idx.at[sl]], t_rows.at[sl], sem))
    for cp in copies:
        cp.wait()

    lanes = lax.iota(jnp.int32, 16)
    colbase = lanes * 16

    def group_body(g, carry):
        for e in range(16):
            b = g * 16 + e
            for c in range(D // 16):
                sl = pl.ds(c * 16, 16)
                d = hr_rows[b, sl] - t_rows[b, sl]
                if c == 0:
                    acc = d * d
                else:
                    acc = acc + d * d
            tbuf[pl.ds(e * 16, 16)] = acc
        tot = jnp.zeros((16,), jnp.float32)
        for k in range(16):
            tot = tot + plsc.load_gather(tbuf, [colbase + k])
        x = tot + 2e-38
        xi = plsc.bitcast(x, jnp.int32)
        y = plsc.bitcast((xi >> 1) + 0x1FBD1DF5, jnp.float32)
        y = 0.5 * (y + x / y)
        y = 0.5 * (y + x / y)
        out_v[pl.ds(pl.multiple_of(g * 16, 16), 16)] = -y
        return carry

    lax.fori_loop(0, GROUPS, group_body, 0)
    pltpu.sync_copy(out_v, out_hbm.at[pl.ds(base, BPW)])


def kernel(heads, relations, tails, entity_table, relation_table):
    ent_a = lax.optimization_barrier(entity_table.T)
    ent_b = lax.optimization_barrier(ent_a)
    rel_lin = lax.optimization_barrier(relation_table.T).T
    hr = _gather_hr_kernel(heads, relations, ent_a.T, rel_lin)
    return _score_kernel(tails, ent_b.T, hr)
